# baseline (speedup 1.0000x reference)
import functools

import jax
import jax.numpy as jnp
from jax import lax
from jax.experimental import pallas as pl
from jax.experimental.pallas import tpu as pltpu
from jax.experimental.pallas import tpu_sc as plsc

N = 10000
R = 4
E = 80000
DF = 128
DH = 128
DE = 128
DQ = 64

NC = 2
NS = 16
WA = 144
HALF = WA // 2
NPAD = 10240
RPT = NPAD // NS
BN = 400
EPT = E // NS
CH = 128
NCHUNK = 42
EPAD = NCHUNK * CH - EPT
ZROW = 64
ZCH = RPT // ZROW
DROW = 128
DCH = RPT // DROW


def _haug_body(x_ref, wf_ref, o_ref):
    h = jnp.tanh(
        lax.dot_general(x_ref[...], wf_ref[...], (((1,), (1,)), ((), ())),
                        preferred_element_type=jnp.float32))
    ones = jnp.ones((BN, 1), jnp.float32)
    zeros = jnp.zeros((BN, WA - DH - 1), jnp.float32)
    o_ref[...] = jnp.concatenate([h, ones, zeros], axis=1)


def _haug(x, w_feat):
    return pl.pallas_call(
        _haug_body,
        grid=(N // BN,),
        in_specs=[
            pl.BlockSpec((BN, DF), lambda i: (i, 0)),
            pl.BlockSpec((DH, DF), lambda i: (0, 0)),
        ],
        out_specs=pl.BlockSpec((BN, WA), lambda i: (i, 0)),
        out_shape=jax.ShapeDtypeStruct((NPAD, WA), jnp.float32),
    )(x, w_feat)


NB = 3


def _seg_body(h_hbm, sidx_hbm, didx_hbm, zeros_hbm, out_hbm,
              rows0, rows1, rows2, sidx, didx, table, acc, sem0, sem1, sem2):
    c = lax.axis_index("c")
    s = lax.axis_index("s")
    bufs = (rows0, rows1, rows2)
    sems = (sem0, sem1, sem2)

    def start_gather(j, b):
        pltpu.make_async_copy(table.at[sidx.at[j]], bufs[b], sems[b]).start()

    def wait_gather(b):
        pltpu.make_async_copy(table.at[sidx.at[0]], bufs[b], sems[b]).wait()

    for p in range(2):
        pltpu.sync_copy(
            h_hbm.at[pl.ds(s * RPT, RPT), pl.ds(p * HALF, HALF)],
            table.at[pl.ds(s * RPT, RPT)])
        plsc.subcore_barrier()
        for i in range(R // NC):
            r = c * (R // NC) + i
            pltpu.sync_copy(zeros_hbm, rows0.at[pl.ds(0, ZROW)])
            for z in range(ZCH):
                row0 = s * RPT + z * ZROW
                pltpu.sync_copy(rows0.at[pl.ds(0, ZROW)],
                                acc.at[pl.ds(row0, ZROW)])
            pltpu.sync_copy(sidx_hbm.at[r, s], sidx)
            pltpu.sync_copy(didx_hbm.at[r, s], didx)
            plsc.subcore_barrier()

            for b in range(NB):
                start_gather(b, b)

            def ring(j, carry):
                for b in range(NB):
                    jj = NB * j + b
                    wait_gather(b)
                    pltpu.sync_copy(bufs[b], acc.at[didx.at[jj]], add=True)
                    start_gather(jj + NB, b)
                return carry

            lax.fori_loop(0, NCHUNK // NB - 1, ring, 0)
            for b in range(NB):
                wait_gather(b)
                pltpu.sync_copy(bufs[b], acc.at[didx.at[NCHUNK - NB + b]],
                                add=True)
            plsc.subcore_barrier()
            for z in range(DCH):
                row0 = s * RPT + z * DROW
                pltpu.sync_copy(acc.at[pl.ds(row0, DROW)],
                                out_hbm.at[p, r, pl.ds(row0, DROW)])


def _segsum(h_aug, sidxp, didxp, zeros_in):
    mesh = plsc.VectorSubcoreMesh(
        core_axis_name="c", subcore_axis_name="s",
        num_cores=NC, num_subcores=NS)
    f = pl.kernel(
        _seg_body,
        out_type=jax.ShapeDtypeStruct((2, R, NPAD, HALF), jnp.float32),
        mesh=mesh,
        scratch_types=[
            pltpu.VMEM((CH, HALF), jnp.float32),
            pltpu.VMEM((CH, HALF), jnp.float32),
            pltpu.VMEM((CH, HALF), jnp.float32),
            pltpu.VMEM((NCHUNK, CH), jnp.int32),
            pltpu.VMEM((NCHUNK, CH), jnp.int32),
            pltpu.VMEM_SHARED((NPAD, HALF), jnp.float32),
            pltpu.VMEM_SHARED((NPAD, HALF), jnp.float32),
            pltpu.SemaphoreType.DMA,
            pltpu.SemaphoreType.DMA,
            pltpu.SemaphoreType.DMA,
        ],
        compiler_params=pltpu.CompilerParams(use_tc_tiling_on_sc=False),
    )
    return f(h_aug, sidxp, didxp, zeros_in)


def _final_body(s_ref, ha_ref, wconv_ref, wattn_ref, b_ref, q_ref, wcat_ref,
                y_ref, a_ref):
    hr = []
    scores = []
    for r in range(R):
        s0 = s_ref[0, r]
        s1 = s_ref[1, r]
        srow = jnp.concatenate([s0, s1[:, :DH - HALF]], axis=1)
        deg = jnp.maximum(s1[:, DH - HALF:DH - HALF + 1], 1.0)
        agg = srow / deg
        h_r = lax.dot_general(agg, wconv_ref[r], (((1,), (1,)), ((), ())),
                              preferred_element_type=jnp.float32)
        t = jnp.tanh(
            lax.dot_general(h_r, wattn_ref[...], (((1,), (1,)), ((), ())),
                            preferred_element_type=jnp.float32)
            + b_ref[...])
        sc = jnp.sum(t * q_ref[...], axis=1, keepdims=True)
        hr.append(h_r)
        scores.append(sc)
    sc = jnp.concatenate(scores, axis=1)
    m = jnp.max(sc, axis=1, keepdims=True)
    ex = jnp.exp(sc - m)
    alpha = ex / jnp.sum(ex, axis=1, keepdims=True)
    h1 = alpha[:, 0:1] * hr[0]
    for r in range(1, R):
        h1 = h1 + alpha[:, r:r + 1] * hr[r]
    h0 = ha_ref[:, :DH]
    wcat = wcat_ref[...]
    y = (lax.dot_general(h0, wcat[:, :DH], (((1,), (1,)), ((), ())),
                         preferred_element_type=jnp.float32)
         + lax.dot_general(h1, wcat[:, DH:], (((1,), (1,)), ((), ())),
                           preferred_element_type=jnp.float32))
    y_ref[...] = y
    a_ref[...] = alpha


def _final(sums, h_aug, w_conv, w_attn, b_attn, q_attn, w_cat):
    return pl.pallas_call(
        _final_body,
        grid=(N // BN,),
        in_specs=[
            pl.BlockSpec((2, R, BN, HALF), lambda i: (0, 0, i, 0)),
            pl.BlockSpec((BN, WA), lambda i: (i, 0)),
            pl.BlockSpec((R, DE, DH), lambda i: (0, 0, 0)),
            pl.BlockSpec((DQ, DE), lambda i: (0, 0)),
            pl.BlockSpec((1, DQ), lambda i: (0, 0)),
            pl.BlockSpec((1, DQ), lambda i: (0, 0)),
            pl.BlockSpec((DE, 2 * DE), lambda i: (0, 0)),
        ],
        out_specs=[
            pl.BlockSpec((BN, DE), lambda i: (i, 0)),
            pl.BlockSpec((BN, R), lambda i: (i, 0)),
        ],
        out_shape=[
            jax.ShapeDtypeStruct((N, DE), jnp.float32),
            jax.ShapeDtypeStruct((N, R), jnp.float32),
        ],
    )(sums, h_aug, w_conv, w_attn, b_attn, q_attn, w_cat)


def kernel(x, edge_index, W_feat, W_conv, W_attn, b_attn, q_attn, W_concat):
    idx = edge_index.reshape(R, 2, NS, EPT)
    pad = jnp.full((R, 2, NS, EPAD), N, dtype=jnp.int32)
    idxp = jnp.concatenate([idx, pad], axis=3)
    sidxp = idxp[:, 0].reshape(R, NS, NCHUNK, CH)
    didxp = idxp[:, 1].reshape(R, NS, NCHUNK, CH)
    zeros_in = jnp.zeros((ZROW, HALF), jnp.float32)

    h_aug = _haug(x, W_feat)
    sums = _segsum(h_aug, sidxp, didxp, zeros_in)
    y, attn = _final(sums, h_aug, W_conv, W_attn,
                     b_attn.reshape(1, DQ), q_attn.reshape(1, DQ), W_concat)
    return (y, attn)


# speedup vs baseline: 1.0500x; 1.0500x over previous
import functools

import jax
import jax.numpy as jnp
from jax import lax
from jax.experimental import pallas as pl
from jax.experimental.pallas import tpu as pltpu
from jax.experimental.pallas import tpu_sc as plsc

N = 10000
R = 4
E = 80000
DF = 128
DH = 128
DE = 128
DQ = 64

NC = 2
NS = 16
WA = 144
HALF = WA // 2
NPAD = 10240
RPT = NPAD // NS
BN = 400
EPT = E // NS
CH = 128
NCHUNK = 40
EPAD = NCHUNK * CH - EPT
ZROW = 64
ZCH = RPT // ZROW
DROW = 128
DCH = RPT // DROW


def _haug_body(x_ref, wf_ref, o_ref):
    h = jnp.tanh(
        lax.dot_general(x_ref[...], wf_ref[...], (((1,), (1,)), ((), ())),
                        preferred_element_type=jnp.float32))
    ones = jnp.ones((BN, 1), jnp.float32)
    zeros = jnp.zeros((BN, WA - DH - 1), jnp.float32)
    o_ref[...] = jnp.concatenate([h, ones, zeros], axis=1)


def _haug(x, w_feat):
    return pl.pallas_call(
        _haug_body,
        grid=(N // BN,),
        in_specs=[
            pl.BlockSpec((BN, DF), lambda i: (i, 0)),
            pl.BlockSpec((DH, DF), lambda i: (0, 0)),
        ],
        out_specs=pl.BlockSpec((BN, WA), lambda i: (i, 0)),
        out_shape=jax.ShapeDtypeStruct((NPAD, WA), jnp.float32),
    )(x, w_feat)


NB = 2


def _seg_body(h_hbm, sidx_hbm, didx_hbm, zeros_hbm, out_hbm,
              rows0, rows1, sidx, didx, table, acc, sem0, sem1):
    c = lax.axis_index("c")
    s = lax.axis_index("s")
    bufs = (rows0, rows1)
    sems = (sem0, sem1)

    def start_gather(j, b):
        pltpu.make_async_copy(table.at[sidx.at[j]], bufs[b], sems[b]).start()

    def wait_gather(b):
        pltpu.make_async_copy(table.at[sidx.at[0]], bufs[b], sems[b]).wait()

    for p in range(2):
        pltpu.sync_copy(
            h_hbm.at[pl.ds(s * RPT, RPT), pl.ds(p * HALF, HALF)],
            table.at[pl.ds(s * RPT, RPT)])
        plsc.subcore_barrier()
        for i in range(R // NC):
            r = c * (R // NC) + i
            pltpu.sync_copy(zeros_hbm, rows0.at[pl.ds(0, ZROW)])
            for z in range(ZCH):
                row0 = s * RPT + z * ZROW
                pltpu.sync_copy(rows0.at[pl.ds(0, ZROW)],
                                acc.at[pl.ds(row0, ZROW)])
            pltpu.sync_copy(sidx_hbm.at[r, s], sidx)
            pltpu.sync_copy(didx_hbm.at[r, s], didx)
            plsc.subcore_barrier()

            for b in range(NB):
                start_gather(b, b)

            def ring(j, carry):
                for b in range(NB):
                    jj = NB * j + b
                    wait_gather(b)
                    pltpu.sync_copy(bufs[b], acc.at[didx.at[jj]], add=True)
                    start_gather(jj + NB, b)
                return carry

            lax.fori_loop(0, NCHUNK // NB - 1, ring, 0)
            for b in range(NB):
                wait_gather(b)
                pltpu.sync_copy(bufs[b], acc.at[didx.at[NCHUNK - NB + b]],
                                add=True)
            plsc.subcore_barrier()
            for z in range(DCH):
                row0 = s * RPT + z * DROW
                pltpu.sync_copy(acc.at[pl.ds(row0, DROW)],
                                out_hbm.at[p, r, pl.ds(row0, DROW)])


def _segsum(h_aug, sidxp, didxp, zeros_in):
    mesh = plsc.VectorSubcoreMesh(
        core_axis_name="c", subcore_axis_name="s",
        num_cores=NC, num_subcores=NS)
    f = pl.kernel(
        _seg_body,
        out_type=jax.ShapeDtypeStruct((2, R, NPAD, HALF), jnp.float32),
        mesh=mesh,
        scratch_types=[
            pltpu.VMEM((CH, HALF), jnp.float32),
            pltpu.VMEM((CH, HALF), jnp.float32),
            pltpu.VMEM((NCHUNK, CH), jnp.int32),
            pltpu.VMEM((NCHUNK, CH), jnp.int32),
            pltpu.VMEM_SHARED((NPAD, HALF), jnp.float32),
            pltpu.VMEM_SHARED((NPAD, HALF), jnp.float32),
            pltpu.SemaphoreType.DMA,
            pltpu.SemaphoreType.DMA,
        ],
        compiler_params=pltpu.CompilerParams(use_tc_tiling_on_sc=False),
    )
    return f(h_aug, sidxp, didxp, zeros_in)


def _final_body(s_ref, ha_ref, wconv_ref, wattn_ref, b_ref, q_ref, wcat_ref,
                y_ref, a_ref):
    hr = []
    scores = []
    for r in range(R):
        s0 = s_ref[0, r]
        s1 = s_ref[1, r]
        rdeg = 1.0 / jnp.maximum(s1[:, DH - HALF:DH - HALF + 1], 1.0)
        wc = wconv_ref[r]
        h_r = (lax.dot_general(s0, wc[:, :HALF], (((1,), (1,)), ((), ())),
                               preferred_element_type=jnp.float32)
               + lax.dot_general(s1[:, :DH - HALF], wc[:, HALF:],
                                 (((1,), (1,)), ((), ())),
                                 preferred_element_type=jnp.float32)) * rdeg
        t = jnp.tanh(
            lax.dot_general(h_r, wattn_ref[...], (((1,), (1,)), ((), ())),
                            preferred_element_type=jnp.float32)
            + b_ref[...])
        sc = jnp.sum(t * q_ref[...], axis=1, keepdims=True)
        hr.append(h_r)
        scores.append(sc)
    sc = jnp.concatenate(scores, axis=1)
    m = jnp.max(sc, axis=1, keepdims=True)
    ex = jnp.exp(sc - m)
    alpha = ex / jnp.sum(ex, axis=1, keepdims=True)
    h1 = alpha[:, 0:1] * hr[0]
    for r in range(1, R):
        h1 = h1 + alpha[:, r:r + 1] * hr[r]
    h0 = ha_ref[:, :DH]
    wcat = wcat_ref[...]
    y = (lax.dot_general(h0, wcat[:, :DH], (((1,), (1,)), ((), ())),
                         preferred_element_type=jnp.float32)
         + lax.dot_general(h1, wcat[:, DH:], (((1,), (1,)), ((), ())),
                           preferred_element_type=jnp.float32))
    y_ref[...] = y
    a_ref[...] = alpha


def _final(sums, h_aug, w_conv, w_attn, b_attn, q_attn, w_cat):
    return pl.pallas_call(
        _final_body,
        grid=(N // BN,),
        in_specs=[
            pl.BlockSpec((2, R, BN, HALF), lambda i: (0, 0, i, 0)),
            pl.BlockSpec((BN, WA), lambda i: (i, 0)),
            pl.BlockSpec((R, DE, DH), lambda i: (0, 0, 0)),
            pl.BlockSpec((DQ, DE), lambda i: (0, 0)),
            pl.BlockSpec((1, DQ), lambda i: (0, 0)),
            pl.BlockSpec((1, DQ), lambda i: (0, 0)),
            pl.BlockSpec((DE, 2 * DE), lambda i: (0, 0)),
        ],
        out_specs=[
            pl.BlockSpec((BN, DE), lambda i: (i, 0)),
            pl.BlockSpec((BN, R), lambda i: (i, 0)),
        ],
        out_shape=[
            jax.ShapeDtypeStruct((N, DE), jnp.float32),
            jax.ShapeDtypeStruct((N, R), jnp.float32),
        ],
    )(sums, h_aug, w_conv, w_attn, b_attn, q_attn, w_cat)


def kernel(x, edge_index, W_feat, W_conv, W_attn, b_attn, q_attn, W_concat):
    idx = edge_index.reshape(R, 2, NS, EPT)
    pad = jnp.full((R, 2, NS, EPAD), N, dtype=jnp.int32)
    idxp = jnp.concatenate([idx, pad], axis=3)
    sidxp = idxp[:, 0].reshape(R, NS, NCHUNK, CH)
    didxp = idxp[:, 1].reshape(R, NS, NCHUNK, CH)
    zeros_in = jnp.zeros((ZROW, HALF), jnp.float32)

    h_aug = _haug(x, W_feat)
    sums = _segsum(h_aug, sidxp, didxp, zeros_in)
    y, attn = _final(sums, h_aug, W_conv, W_attn,
                     b_attn.reshape(1, DQ), q_attn.reshape(1, DQ), W_concat)
    return (y, attn)


# speedup vs baseline: 1.0509x; 1.0008x over previous
import jax
import jax.numpy as jnp
from jax import lax
from jax.experimental import pallas as pl
from jax.experimental.pallas import tpu as pltpu
from jax.experimental.pallas import tpu_sc as plsc

N = 10000
R = 4
E = 80000
DF = 128
DH = 128
DE = 128
DQ = 64

NC = 2
NS = 16
WA = 144
HALF = WA // 2
NPAD = 10240
RPT = NPAD // NS
BN = 400
EPT = E // NS
CH = 128
NCHUNK = 40
EPAD = NCHUNK * CH - EPT
ZROW = 64
ZCH = RPT // ZROW
DROW = 128
DCH = RPT // DROW


def _haug_body(x_ref, wf_ref, o_ref):
    h = jnp.tanh(
        lax.dot_general(x_ref[...], wf_ref[...], (((1,), (1,)), ((), ())),
                        preferred_element_type=jnp.float32))
    ones = jnp.ones((BN, 1), jnp.float32)
    zeros = jnp.zeros((BN, WA - DH - 1), jnp.float32)
    o_ref[...] = jnp.concatenate([h, ones, zeros], axis=1)


def _haug(x, w_feat):
    return pl.pallas_call(
        _haug_body,
        grid=(N // BN,),
        in_specs=[
            pl.BlockSpec((BN, DF), lambda i: (i, 0)),
            pl.BlockSpec((DH, DF), lambda i: (0, 0)),
        ],
        out_specs=pl.BlockSpec((BN, WA), lambda i: (i, 0)),
        out_shape=jax.ShapeDtypeStruct((NPAD, WA), jnp.float32),
    )(x, w_feat)


NB = 2


def _seg_body(h_hbm, sidx_hbm, didx_hbm, zeros_hbm, out_hbm,
              rows0, rows1, sidx, didx, table, acc, sem0, sem1):
    c = lax.axis_index("c")
    s = lax.axis_index("s")
    bufs = (rows0, rows1)
    sems = (sem0, sem1)

    def start_gather(j, b):
        pltpu.make_async_copy(table.at[sidx.at[j]], bufs[b], sems[b]).start()

    def wait_gather(b):
        pltpu.make_async_copy(table.at[sidx.at[0]], bufs[b], sems[b]).wait()

    for p in range(2):
        pltpu.sync_copy(
            h_hbm.at[pl.ds(s * RPT, RPT), pl.ds(p * HALF, HALF)],
            table.at[pl.ds(s * RPT, RPT)])
        plsc.subcore_barrier()
        for i in range(R // NC):
            r = c * (R // NC) + i
            pltpu.sync_copy(zeros_hbm, rows0.at[pl.ds(0, ZROW)])
            for z in range(ZCH):
                row0 = s * RPT + z * ZROW
                pltpu.sync_copy(rows0.at[pl.ds(0, ZROW)],
                                acc.at[pl.ds(row0, ZROW)])
            pltpu.sync_copy(sidx_hbm.at[r, s], sidx)
            pltpu.sync_copy(didx_hbm.at[r, s], didx)
            plsc.subcore_barrier()

            for b in range(NB):
                start_gather(b, b)

            def ring(j, carry):
                for b in range(NB):
                    jj = NB * j + b
                    wait_gather(b)
                    pltpu.sync_copy(bufs[b], acc.at[didx.at[jj]], add=True)
                    start_gather(jj + NB, b)
                return carry

            lax.fori_loop(0, NCHUNK // NB - 1, ring, 0)
            for b in range(NB):
                wait_gather(b)
                pltpu.sync_copy(bufs[b], acc.at[didx.at[NCHUNK - NB + b]],
                                add=True)
            plsc.subcore_barrier()
            for z in range(DCH):
                row0 = s * RPT + z * DROW
                pltpu.sync_copy(acc.at[pl.ds(row0, DROW)],
                                out_hbm.at[p, r, pl.ds(row0, DROW)])


def _segsum(h_aug, sidxp, didxp, zeros_in):
    mesh = plsc.VectorSubcoreMesh(
        core_axis_name="c", subcore_axis_name="s",
        num_cores=NC, num_subcores=NS)
    f = pl.kernel(
        _seg_body,
        out_type=jax.ShapeDtypeStruct((2, R, NPAD, HALF), jnp.float32),
        mesh=mesh,
        scratch_types=[
            pltpu.VMEM((CH, HALF), jnp.float32),
            pltpu.VMEM((CH, HALF), jnp.float32),
            pltpu.VMEM((NCHUNK, CH), jnp.int32),
            pltpu.VMEM((NCHUNK, CH), jnp.int32),
            pltpu.VMEM_SHARED((NPAD, HALF), jnp.float32),
            pltpu.VMEM_SHARED((NPAD, HALF), jnp.float32),
            pltpu.SemaphoreType.DMA,
            pltpu.SemaphoreType.DMA,
        ],
        compiler_params=pltpu.CompilerParams(use_tc_tiling_on_sc=False),
    )
    return f(h_aug, sidxp, didxp, zeros_in)


def _final_body(s_ref, ha_ref, wconv_ref, wattn_ref, b_ref, q_ref, wcat_ref,
                y_ref, a_ref):
    hr = []
    scores = []
    for r in range(R):
        s0 = s_ref[0, r]
        s1 = s_ref[1, r]
        rdeg = 1.0 / jnp.maximum(s1[:, DH - HALF:DH - HALF + 1], 1.0)
        wc = wconv_ref[r]
        h_r = (lax.dot_general(s0, wc[:, :HALF], (((1,), (1,)), ((), ())),
                               preferred_element_type=jnp.float32)
               + lax.dot_general(s1[:, :DH - HALF], wc[:, HALF:],
                                 (((1,), (1,)), ((), ())),
                                 preferred_element_type=jnp.float32)) * rdeg
        t = jnp.tanh(
            lax.dot_general(h_r, wattn_ref[...], (((1,), (1,)), ((), ())),
                            preferred_element_type=jnp.float32)
            + b_ref[...])
        sc = jnp.sum(t * q_ref[...], axis=1, keepdims=True)
        hr.append(h_r)
        scores.append(sc)
    sc = jnp.concatenate(scores, axis=1)
    m = jnp.max(sc, axis=1, keepdims=True)
    ex = jnp.exp(sc - m)
    alpha = ex / jnp.sum(ex, axis=1, keepdims=True)
    h1 = alpha[:, 0:1] * hr[0]
    for r in range(1, R):
        h1 = h1 + alpha[:, r:r + 1] * hr[r]
    h0 = ha_ref[:, :DH]
    wcat = wcat_ref[...]
    y = (lax.dot_general(h0, wcat[:, :DH], (((1,), (1,)), ((), ())),
                         preferred_element_type=jnp.float32)
         + lax.dot_general(h1, wcat[:, DH:], (((1,), (1,)), ((), ())),
                           preferred_element_type=jnp.float32))
    y_ref[...] = y
    a_ref[...] = alpha


def _final(sums, h_aug, w_conv, w_attn, b_attn, q_attn, w_cat):
    return pl.pallas_call(
        _final_body,
        grid=(N // BN,),
        in_specs=[
            pl.BlockSpec((2, R, BN, HALF), lambda i: (0, 0, i, 0)),
            pl.BlockSpec((BN, WA), lambda i: (i, 0)),
            pl.BlockSpec((R, DE, DH), lambda i: (0, 0, 0)),
            pl.BlockSpec((DQ, DE), lambda i: (0, 0)),
            pl.BlockSpec((1, DQ), lambda i: (0, 0)),
            pl.BlockSpec((1, DQ), lambda i: (0, 0)),
            pl.BlockSpec((DE, 2 * DE), lambda i: (0, 0)),
        ],
        out_specs=[
            pl.BlockSpec((BN, DE), lambda i: (i, 0)),
            pl.BlockSpec((BN, R), lambda i: (i, 0)),
        ],
        out_shape=[
            jax.ShapeDtypeStruct((N, DE), jnp.float32),
            jax.ShapeDtypeStruct((N, R), jnp.float32),
        ],
    )(sums, h_aug, w_conv, w_attn, b_attn, q_attn, w_cat)


def kernel(x, edge_index, W_feat, W_conv, W_attn, b_attn, q_attn, W_concat):
    idx = edge_index.reshape(R, 2, NS, EPT)
    pad = jnp.full((R, 2, NS, EPAD), N, dtype=jnp.int32)
    idxp = jnp.concatenate([idx, pad], axis=3)
    sidxp = idxp[:, 0].reshape(R, NS, NCHUNK, CH)
    didxp = idxp[:, 1].reshape(R, NS, NCHUNK, CH)
    zeros_in = jnp.zeros((ZROW, HALF), jnp.float32)

    h_aug = _haug(x, W_feat)
    sums = _segsum(h_aug, sidxp, didxp, zeros_in)
    y, attn = _final(sums, h_aug, W_conv, W_attn,
                     b_attn.reshape(1, DQ), q_attn.reshape(1, DQ), W_concat)
    return (y, attn)
